# baseline (speedup 1.0000x reference)
import jax
import jax.numpy as jnp
from jax.experimental import pallas as pl

B_TILE = 8


def _onehot_block(t_ref, out_ref):
    tb = t_ref[...]
    cls = jax.lax.broadcasted_iota(jnp.int32, out_ref.shape, 1)
    out_ref[...] = (tb[:, None, :] == cls).astype(jnp.float32)


def kernel(t, ones):
    B, S = t.shape
    C = ones.shape[0]
    t32 = t.astype(jnp.int32)
    out = pl.pallas_call(
        _onehot_block,
        grid=(B // B_TILE,),
        in_specs=[pl.BlockSpec((B_TILE, S), lambda i: (i, 0))],
        out_specs=pl.BlockSpec((B_TILE, C, S), lambda i: (i, 0, 0)),
        out_shape=jax.ShapeDtypeStruct((B, C, S), jnp.float32),
    )(t32)
    return out
